# Initial kernel scaffold; baseline (speedup 1.0000x reference)
import jax
import jax.numpy as jnp
from jax.experimental import pallas as pl


def kernel(x, weight, bias):
    raise NotImplementedError("write your pallas kernel here")


# speedup vs baseline: 1.6601x; 1.6601x over previous
import numpy as np

import jax
import jax.numpy as jnp
from jax.experimental import pallas as pl
from jax.experimental.pallas import tpu as pltpu


def _up_matrix(n_in: int, n_out: int) -> np.ndarray:
    dst = np.arange(n_out, dtype=np.float64)
    src = np.maximum((dst + 0.5) * (n_in / n_out) - 0.5, 0.0)
    i0 = np.minimum(np.floor(src).astype(np.int64), n_in - 1)
    i1 = np.minimum(i0 + 1, n_in - 1)
    frac = (src - i0).astype(np.float32)
    rows = np.arange(n_out)
    m = np.zeros((n_out, n_in), np.float32)
    m[rows, i0] += 1.0 - frac
    m[rows, i1] += frac
    return m


def _fused_kernel(x_ref, w_ref, b_ref, uh_ref, uwt_ref, o_ref):
    ho, h = uh_ref.shape
    w, wo = uwt_ref.shape
    cout = w_ref.shape[0]

    xb = x_ref[0].astype(jnp.bfloat16)
    y = jnp.dot(w_ref[...], xb, preferred_element_type=jnp.float32)
    y = y + b_ref[...]

    y3 = y.reshape(cout, h, w).astype(jnp.bfloat16)
    uh_b = jnp.broadcast_to(uh_ref[...][None, :, :], (cout, ho, h))
    z = jnp.einsum("cih,chw->ciw", uh_b, y3,
                   preferred_element_type=jnp.float32)

    z2 = z.astype(jnp.bfloat16).reshape(cout * ho, w)
    out = jnp.dot(z2, uwt_ref[...], preferred_element_type=jnp.float32)
    o_ref[0] = out.reshape(cout, ho, wo)


def kernel(x, weight, bias):
    B, Cin, H, W = x.shape
    if weight.ndim == 4:
        weight = weight.reshape(weight.shape[0], weight.shape[1])
    Cout = weight.shape[0]
    Ho, Wo = 2 * H, 2 * W

    uh = jnp.asarray(_up_matrix(H, Ho)).astype(jnp.bfloat16)
    uwt = jnp.asarray(_up_matrix(W, Wo).T).astype(jnp.bfloat16)
    w2 = weight.astype(jnp.bfloat16)
    b2 = bias.astype(jnp.float32).reshape(Cout, 1)

    params = pltpu.CompilerParams(
        dimension_semantics=("parallel",),
        vmem_limit_bytes=64 * 1024 * 1024,
    )

    out = pl.pallas_call(
        _fused_kernel,
        out_shape=jax.ShapeDtypeStruct((B, Cout, Ho, Wo), x.dtype),
        grid=(B,),
        in_specs=[
            pl.BlockSpec((1, Cin, H * W), lambda b: (b, 0, 0)),
            pl.BlockSpec((Cout, Cin), lambda b: (0, 0)),
            pl.BlockSpec((Cout, 1), lambda b: (0, 0)),
            pl.BlockSpec((Ho, H), lambda b: (0, 0)),
            pl.BlockSpec((W, Wo), lambda b: (0, 0)),
        ],
        out_specs=pl.BlockSpec((1, Cout, Ho, Wo), lambda b: (b, 0, 0, 0)),
        compiler_params=params,
    )(x.reshape(B, Cin, H * W), w2, b2, uh, uwt)

    return out


# speedup vs baseline: 2.4173x; 1.4561x over previous
import numpy as np

import jax
import jax.numpy as jnp
from jax.experimental import pallas as pl
from jax.experimental.pallas import tpu as pltpu


def _up_matrix(n_in: int, n_out: int) -> np.ndarray:
    dst = np.arange(n_out, dtype=np.float64)
    src = np.maximum((dst + 0.5) * (n_in / n_out) - 0.5, 0.0)
    i0 = np.minimum(np.floor(src).astype(np.int64), n_in - 1)
    i1 = np.minimum(i0 + 1, n_in - 1)
    frac = (src - i0).astype(np.float32)
    rows = np.arange(n_out)
    m = np.zeros((n_out, n_in), np.float32)
    m[rows, i0] += 1.0 - frac
    m[rows, i1] += frac
    return m


def _fused_kernel(x_ref, w_ref, b_ref, uh_ref, uwt_ref, o_ref):
    ho, h = uh_ref.shape
    w, wo = uwt_ref.shape
    cout = w_ref.shape[0]
    cin = x_ref.shape[1]

    xb = x_ref[0].astype(jnp.bfloat16).reshape(cin, h * w)
    y = jnp.dot(w_ref[...], xb, preferred_element_type=jnp.float32)
    y = y + b_ref[...]

    y3 = y.reshape(cout, h, w).astype(jnp.bfloat16)
    uh_b = jnp.broadcast_to(uh_ref[...][None, :, :], (cout, ho, h))
    z = jnp.einsum("cih,chw->ciw", uh_b, y3,
                   preferred_element_type=jnp.float32)

    uwt_b = jnp.broadcast_to(uwt_ref[...][None, :, :], (cout, w, wo))
    o = jnp.einsum("ciw,cwv->civ", z.astype(jnp.bfloat16), uwt_b,
                   preferred_element_type=jnp.float32)
    o_ref[0] = o


def kernel(x, weight, bias):
    B, Cin, H, W = x.shape
    if weight.ndim == 4:
        weight = weight.reshape(weight.shape[0], weight.shape[1])
    Cout = weight.shape[0]
    Ho, Wo = 2 * H, 2 * W

    uh = jnp.asarray(_up_matrix(H, Ho)).astype(jnp.bfloat16)
    uwt = jnp.asarray(_up_matrix(W, Wo).T).astype(jnp.bfloat16)
    w2 = weight.astype(jnp.bfloat16)
    b2 = bias.astype(jnp.float32).reshape(Cout, 1)

    params = pltpu.CompilerParams(
        dimension_semantics=("parallel",),
        vmem_limit_bytes=64 * 1024 * 1024,
    )

    out = pl.pallas_call(
        _fused_kernel,
        out_shape=jax.ShapeDtypeStruct((B, Cout, Ho, Wo), x.dtype),
        grid=(B,),
        in_specs=[
            pl.BlockSpec((1, Cin, H, W), lambda b: (b, 0, 0, 0)),
            pl.BlockSpec((Cout, Cin), lambda b: (0, 0)),
            pl.BlockSpec((Cout, 1), lambda b: (0, 0)),
            pl.BlockSpec((Ho, H), lambda b: (0, 0)),
            pl.BlockSpec((W, Wo), lambda b: (0, 0)),
        ],
        out_specs=pl.BlockSpec((1, Cout, Ho, Wo), lambda b: (b, 0, 0, 0)),
        compiler_params=params,
    )(x, w2, b2, uh, uwt)

    return out


# speedup vs baseline: 2.4485x; 1.0129x over previous
import numpy as np

import jax
import jax.numpy as jnp
from jax.experimental import pallas as pl
from jax.experimental.pallas import tpu as pltpu


def _up_matrix(n_in: int, n_out: int) -> np.ndarray:
    dst = np.arange(n_out, dtype=np.float64)
    src = np.maximum((dst + 0.5) * (n_in / n_out) - 0.5, 0.0)
    i0 = np.minimum(np.floor(src).astype(np.int64), n_in - 1)
    i1 = np.minimum(i0 + 1, n_in - 1)
    frac = (src - i0).astype(np.float32)
    rows = np.arange(n_out)
    m = np.zeros((n_out, n_in), np.float32)
    m[rows, i0] += 1.0 - frac
    m[rows, i1] += frac
    return m


def _fused_kernel(x_ref, w_ref, b_ref, uh_ref, uwt_ref, o_ref):
    cout, ho, h = uh_ref.shape
    w, wo = uwt_ref.shape[1:]
    cin = x_ref.shape[1]

    xb = x_ref[0].astype(jnp.bfloat16).reshape(cin, h * w)
    y = jnp.dot(w_ref[...], xb, preferred_element_type=jnp.float32)
    y = (y + b_ref[...]).astype(jnp.bfloat16)

    y3 = y.reshape(cout, h, w)
    z = jnp.einsum("cih,chw->ciw", uh_ref[...], y3,
                   preferred_element_type=jnp.float32)

    o = jnp.einsum("ciw,cwv->civ", z.astype(jnp.bfloat16), uwt_ref[...],
                   preferred_element_type=jnp.float32)
    o_ref[0] = o


def kernel(x, weight, bias):
    B, Cin, H, W = x.shape
    if weight.ndim == 4:
        weight = weight.reshape(weight.shape[0], weight.shape[1])
    Cout = weight.shape[0]
    Ho, Wo = 2 * H, 2 * W

    uh = jnp.asarray(
        np.broadcast_to(_up_matrix(H, Ho), (Cout, Ho, H))
    ).astype(jnp.bfloat16)
    uwt = jnp.asarray(
        np.broadcast_to(_up_matrix(W, Wo).T, (Cout, W, Wo))
    ).astype(jnp.bfloat16)
    w2 = weight.astype(jnp.bfloat16)
    b2 = bias.astype(jnp.float32).reshape(Cout, 1)

    params = pltpu.CompilerParams(
        dimension_semantics=("parallel",),
        vmem_limit_bytes=64 * 1024 * 1024,
    )

    out = pl.pallas_call(
        _fused_kernel,
        out_shape=jax.ShapeDtypeStruct((B, Cout, Ho, Wo), x.dtype),
        grid=(B,),
        in_specs=[
            pl.BlockSpec((1, Cin, H, W), lambda b: (b, 0, 0, 0)),
            pl.BlockSpec((Cout, Cin), lambda b: (0, 0)),
            pl.BlockSpec((Cout, 1), lambda b: (0, 0)),
            pl.BlockSpec((Cout, Ho, H), lambda b: (0, 0, 0)),
            pl.BlockSpec((Cout, W, Wo), lambda b: (0, 0, 0)),
        ],
        out_specs=pl.BlockSpec((1, Cout, Ho, Wo), lambda b: (b, 0, 0, 0)),
        compiler_params=params,
    )(x, w2, b2, uh, uwt)

    return out
